# baseline (speedup 1.0000x reference)
import jax
import jax.numpy as jnp
from jax.experimental import pallas as pl
from jax.experimental.pallas import tpu as pltpu

_NUM_Q = 8
_K = 1024
_DIM = 256
_TB = 512
_SPLIT = 4


def _rvq_body(cb_ref, x_ref, qout_ref, idx_ref, loss_ref, cnorm_ref,
              cbh_ref, cbm_ref, cbl_ref):
    @pl.when(pl.program_id(0) == 0)
    def _init():
        cb3 = cb_ref[...]
        cnorm_ref[...] = jnp.sum(cb3 * cb3, axis=-1)
        loss_ref[...] = jnp.zeros_like(loss_ref)
        hi = cb3.astype(jnp.bfloat16).astype(jnp.float32)
        r1 = cb3 - hi
        mid = r1.astype(jnp.bfloat16).astype(jnp.float32)
        lo = (r1 - mid).astype(jnp.bfloat16).astype(jnp.float32)
        cbh_ref[...] = hi
        cbm_ref[...] = mid
        cbl_ref[...] = lo

    def layer_step(q, residual, qout):
        cb = cb_ref[q]
        dots = jax.lax.dot_general(
            residual, cb, (((1,), (1,)), ((), ())),
            preferred_element_type=jnp.float32,
            precision=jax.lax.Precision.DEFAULT)
        rnorm = jnp.sum(residual * residual, axis=1, keepdims=True)
        d = rnorm - 2.0 * dots + cnorm_ref[q:q + 1, :]
        dmin = jnp.min(d, axis=1, keepdims=True)
        iota = jax.lax.broadcasted_iota(jnp.int32, d.shape, 1)
        idx = jnp.min(jnp.where(d == dmin, iota, _K), axis=1,
                      keepdims=True)
        onehot = (iota == idx).astype(jnp.float32)
        dn = (((1,), (0,)), ((), ()))
        quant = (jax.lax.dot_general(
                     onehot, cbh_ref[q], dn,
                     preferred_element_type=jnp.float32,
                     precision=jax.lax.Precision.DEFAULT)
                 + jax.lax.dot_general(
                     onehot, cbm_ref[q], dn,
                     preferred_element_type=jnp.float32,
                     precision=jax.lax.Precision.DEFAULT)
                 + jax.lax.dot_general(
                     onehot, cbl_ref[q], dn,
                     preferred_element_type=jnp.float32,
                     precision=jax.lax.Precision.DEFAULT))
        return residual - quant, qout + quant, idx, jnp.sum(rnorm)

    h = _TB // _SPLIT
    res = [x_ref[s * h:(s + 1) * h, :] for s in range(_SPLIT)]
    qo = [jnp.zeros((h, _DIM), jnp.float32) for _ in range(_SPLIT)]
    idx_cols = [[] for _ in range(_SPLIT)]
    loss_parts = [[] for _ in range(_SPLIT)]
    for q in range(_NUM_Q):
        for s in range(_SPLIT):
            res[s], qo[s], idx, rn = layer_step(q, res[s], qo[s])
            idx_cols[s].append(idx)
            if q > 0:
                loss_parts[s].append(rn)
    for s in range(_SPLIT):
        loss_parts[s].append(jnp.sum(res[s] * res[s]))
        qout_ref[s * h:(s + 1) * h, :] = qo[s]
        idx_ref[s * h:(s + 1) * h, :] = jnp.concatenate(idx_cols[s], axis=1)
    scale = 1.25 / float(16 * 1024 * _DIM)
    totals = [sum(parts[q] for parts in loss_parts) * scale
              for q in range(_NUM_Q)]
    loss_ref[...] += jnp.stack(
        [jnp.broadcast_to(t, (128,)) for t in totals])


def kernel(x, codebooks):
    b, t, dim = x.shape
    ntok = b * t
    x2 = x.reshape(ntok, dim)
    qout2, idx_t, loss_mat = pl.pallas_call(
        _rvq_body,
        grid=(ntok // _TB,),
        in_specs=[
            pl.BlockSpec((_NUM_Q, _K, _DIM), lambda i: (0, 0, 0)),
            pl.BlockSpec((_TB, _DIM), lambda i: (i, 0)),
        ],
        out_specs=[
            pl.BlockSpec((_TB, _DIM), lambda i: (i, 0)),
            pl.BlockSpec((_TB, _NUM_Q), lambda i: (i, 0)),
            pl.BlockSpec((_NUM_Q, 128), lambda i: (0, 0)),
        ],
        out_shape=[
            jax.ShapeDtypeStruct((ntok, dim), jnp.float32),
            jax.ShapeDtypeStruct((ntok, _NUM_Q), jnp.int32),
            jax.ShapeDtypeStruct((_NUM_Q, 128), jnp.float32),
        ],
        scratch_shapes=[
            pltpu.VMEM((_NUM_Q, _K), jnp.float32),
            pltpu.VMEM((_NUM_Q, _K, _DIM), jnp.float32),
            pltpu.VMEM((_NUM_Q, _K, _DIM), jnp.float32),
            pltpu.VMEM((_NUM_Q, _K, _DIM), jnp.float32),
        ],
    )(codebooks, x2)
    quantized = qout2.reshape(b, t, dim)
    indices = idx_t.T.reshape(_NUM_Q, b, t)
    losses = loss_mat[:, 0]
    return quantized, indices, losses


# speedup vs baseline: 1.5327x; 1.5327x over previous
import jax
import jax.numpy as jnp
from jax.experimental import pallas as pl
from jax.experimental.pallas import tpu as pltpu

_NUM_Q = 8
_K = 1024
_DIM = 256
_TB = 1024
_SPLIT = 2


def _rvq_body(cb_ref, x_ref, qout_ref, idx_ref, loss_ref, cnorm_ref,
              cbh_ref, cbm_ref, cbl_ref):
    @pl.when(pl.program_id(0) == 0)
    def _init():
        cb3 = cb_ref[...]
        cnorm_ref[...] = jnp.sum(cb3 * cb3, axis=-1)
        loss_ref[...] = jnp.zeros_like(loss_ref)
        hi = cb3.astype(jnp.bfloat16).astype(jnp.float32)
        r1 = cb3 - hi
        mid = r1.astype(jnp.bfloat16).astype(jnp.float32)
        lo = (r1 - mid).astype(jnp.bfloat16).astype(jnp.float32)
        cbh_ref[...] = hi
        cbm_ref[...] = mid
        cbl_ref[...] = lo

    def layer_step(q, residual, qout):
        cb = cb_ref[q]
        dots = jax.lax.dot_general(
            residual, cb, (((1,), (1,)), ((), ())),
            preferred_element_type=jnp.float32,
            precision=jax.lax.Precision.DEFAULT)
        rnorm = jnp.sum(residual * residual, axis=1, keepdims=True)
        d = rnorm - 2.0 * dots + cnorm_ref[q:q + 1, :]
        dmin = jnp.min(d, axis=1, keepdims=True)
        iota = jax.lax.broadcasted_iota(jnp.int32, d.shape, 1)
        idx = jnp.min(jnp.where(d == dmin, iota, _K), axis=1,
                      keepdims=True)
        onehot = (iota == idx).astype(jnp.float32)
        dn = (((1,), (0,)), ((), ()))
        quant = (jax.lax.dot_general(
                     onehot, cbh_ref[q], dn,
                     preferred_element_type=jnp.float32,
                     precision=jax.lax.Precision.DEFAULT)
                 + jax.lax.dot_general(
                     onehot, cbm_ref[q], dn,
                     preferred_element_type=jnp.float32,
                     precision=jax.lax.Precision.DEFAULT)
                 + jax.lax.dot_general(
                     onehot, cbl_ref[q], dn,
                     preferred_element_type=jnp.float32,
                     precision=jax.lax.Precision.DEFAULT))
        return residual - quant, qout + quant, idx, jnp.sum(rnorm)

    h = _TB // _SPLIT
    res = [x_ref[s * h:(s + 1) * h, :] for s in range(_SPLIT)]
    qo = [jnp.zeros((h, _DIM), jnp.float32) for _ in range(_SPLIT)]
    idx_cols = [[] for _ in range(_SPLIT)]
    loss_parts = [[] for _ in range(_SPLIT)]
    for q in range(_NUM_Q):
        for s in range(_SPLIT):
            res[s], qo[s], idx, rn = layer_step(q, res[s], qo[s])
            idx_cols[s].append(idx)
            if q > 0:
                loss_parts[s].append(rn)
    for s in range(_SPLIT):
        loss_parts[s].append(jnp.sum(res[s] * res[s]))
        qout_ref[s * h:(s + 1) * h, :] = qo[s]
        idx_ref[s * h:(s + 1) * h, :] = jnp.concatenate(idx_cols[s], axis=1)
    scale = 1.25 / float(16 * 1024 * _DIM)
    totals = [sum(parts[q] for parts in loss_parts) * scale
              for q in range(_NUM_Q)]
    loss_ref[...] += jnp.stack(
        [jnp.broadcast_to(t, (128,)) for t in totals])


def kernel(x, codebooks):
    b, t, dim = x.shape
    ntok = b * t
    x2 = x.reshape(ntok, dim)
    qout2, idx_t, loss_mat = pl.pallas_call(
        _rvq_body,
        grid=(ntok // _TB,),
        in_specs=[
            pl.BlockSpec((_NUM_Q, _K, _DIM), lambda i: (0, 0, 0)),
            pl.BlockSpec((_TB, _DIM), lambda i: (i, 0)),
        ],
        out_specs=[
            pl.BlockSpec((_TB, _DIM), lambda i: (i, 0)),
            pl.BlockSpec((_TB, _NUM_Q), lambda i: (i, 0)),
            pl.BlockSpec((_NUM_Q, 128), lambda i: (0, 0)),
        ],
        out_shape=[
            jax.ShapeDtypeStruct((ntok, dim), jnp.float32),
            jax.ShapeDtypeStruct((ntok, _NUM_Q), jnp.int32),
            jax.ShapeDtypeStruct((_NUM_Q, 128), jnp.float32),
        ],
        scratch_shapes=[
            pltpu.VMEM((_NUM_Q, _K), jnp.float32),
            pltpu.VMEM((_NUM_Q, _K, _DIM), jnp.float32),
            pltpu.VMEM((_NUM_Q, _K, _DIM), jnp.float32),
            pltpu.VMEM((_NUM_Q, _K, _DIM), jnp.float32),
        ],
    )(codebooks, x2)
    quantized = qout2.reshape(b, t, dim)
    indices = idx_t.T.reshape(_NUM_Q, b, t)
    losses = loss_mat[:, 0]
    return quantized, indices, losses
